# baseline (speedup 1.0000x reference)
import functools

import jax
import jax.numpy as jnp
from jax import lax
from jax.experimental import pallas as pl
from jax.experimental.pallas import tpu as pltpu
from jax.experimental.pallas import tpu_sc as plsc

B = 16384
F = 26
D = 16
NT = F + 2
CONCAT = NT * D
HID = 256
FIELD_VOCAB = 100000

NC, NS = 2, 16
NW = NC * NS
BPW = B // NW
CH = 128
NCH = BPW // CH
L = 16
RPM = 128 // D


def _gather_body(idx_hbm, u8, i8, f8, out_hbm,
                 idx_v, mi_v0, mi_v1, mrg_v0, mrg_v1, big_v, isem, gsem):
    mi_v = (mi_v0, mi_v1)
    mrg_v = (mrg_v0, mrg_v1)
    wid = lax.axis_index("s") * NC + lax.axis_index("c")
    base = wid * BPW

    icps = [
        pltpu.async_copy(
            idx_hbm.at[:, pl.ds(base + c * CH, CH)],
            idx_v.at[pl.ds(c * 32, 32), :], isem)
        for c in range(NCH)
    ]
    for cp in icps:
        cp.wait()

    def tbl(f):
        return u8 if f == 0 else (i8 if f == 1 else f8)

    def ld_idx(c, f, g):
        lanes = lax.iota(jnp.int32, L)
        return plsc.load_gather(
            idx_v, [jnp.full((L,), c * 32 + f, jnp.int32), g * L + lanes])

    def stage_indices(c, f):
        def sg(g, _):
            v = ld_idx(c, f, g)
            lanes = lax.iota(jnp.int32, L)
            plsc.store_scatter(mi_v[f % 2], [g * L + lanes], v >> 3)
            return 0

        lax.fori_loop(0, CH // L, sg, 0)

    def fire(c, f):
        return pltpu.async_copy(
            tbl(f).at[mi_v[f % 2]], mrg_v[f % 2], gsem)

    def extract(c, f):
        def eg(g, _):
            v = ld_idx(c, f, g)
            sub = (v & 7) * D
            rows = lax.iota(jnp.int32, L) + g * L
            for k in range(D):
                vals = plsc.load_gather(mrg_v[f % 2], [rows, sub + k])
                plsc.store_scatter(
                    big_v, [rows, jnp.full((L,), f * D + k, jnp.int32)],
                    vals)
            return 0

        lax.fori_loop(0, CH // L, eg, 0)

    def chunk(c, _):
        stage_indices(c, 0)
        cp0 = fire(c, 0)
        for f in range(NT):
            if f + 1 < NT:
                stage_indices(c, f + 1)
                cp1 = fire(c, f + 1)
            cp0.wait()
            extract(c, f)
            if f + 1 < NT:
                cp0 = cp1
        pltpu.sync_copy(big_v, out_hbm.at[pl.ds(base + c * CH, CH)])
        return 0

    lax.fori_loop(0, NCH, chunk, 0)


@functools.partial(
    pl.kernel,
    out_type=jax.ShapeDtypeStruct((B, CONCAT), jnp.float32),
    mesh=plsc.VectorSubcoreMesh(core_axis_name="c", subcore_axis_name="s"),
    scratch_types=[
        pltpu.VMEM((NCH * 32, CH), jnp.int32),
        pltpu.VMEM((CH,), jnp.int32),
        pltpu.VMEM((CH,), jnp.int32),
        pltpu.VMEM((CH, 128), jnp.float32),
        pltpu.VMEM((CH, 128), jnp.float32),
        pltpu.VMEM((CH, CONCAT), jnp.float32),
        pltpu.SemaphoreType.DMA,
        pltpu.SemaphoreType.DMA,
    ],
    compiler_params=pltpu.CompilerParams(needs_layout_passes=False),
)
def _gather_all(idx, u8, i8, f8, out,
                idx_v, mi_v0, mi_v1, mrg_v0, mrg_v1, big_v, isem, gsem):
    _gather_body(idx, u8, i8, f8, out,
                 idx_v, mi_v0, mi_v1, mrg_v0, mrg_v1, big_v, isem, gsem)


BLK = 512


def _mlp_body(x_ref, w1t_ref, b1_ref, w2_ref, b2_ref, o_ref):
    x = x_ref[...]
    h = jnp.dot(x, w1t_ref[...], preferred_element_type=jnp.float32)
    h = jnp.maximum(h + b1_ref[...], 0.0)
    d = jnp.sum(h * w2_ref[...], axis=1, keepdims=True)
    fm = jnp.sum(x, axis=1, keepdims=True)
    z = fm + d + b2_ref[...]
    o_ref[...] = 10.0 / (1.0 + jnp.exp(-z))


def _mlp(fm_terms, w1t, b1, w2, b2):
    return pl.pallas_call(
        _mlp_body,
        grid=(B // BLK,),
        in_specs=[
            pl.BlockSpec((BLK, CONCAT), lambda i: (i, 0)),
            pl.BlockSpec((CONCAT, HID), lambda i: (0, 0)),
            pl.BlockSpec((1, HID), lambda i: (0, 0)),
            pl.BlockSpec((1, HID), lambda i: (0, 0)),
            pl.BlockSpec((1, 1), lambda i: (0, 0)),
        ],
        out_specs=pl.BlockSpec((BLK, 1), lambda i: (i, 0)),
        out_shape=jax.ShapeDtypeStruct((B, 1), jnp.float32),
    )(fm_terms, w1t, b1, w2, b2)


def kernel(user, item, feature, user_table, item_table, feat_tables,
           W1, b1, W2, b2):
    offs = jnp.arange(F, dtype=jnp.int32) * FIELD_VOCAB
    idx2 = jnp.concatenate(
        [user.astype(jnp.int32)[None],
         item.astype(jnp.int32)[None],
         (feature.astype(jnp.int32) + offs[None, :]).T,
         jnp.zeros((32 - NT, B), jnp.int32)], axis=0)
    u8 = user_table.reshape(-1, 128)
    i8 = item_table.reshape(-1, 128)
    f8 = feat_tables.reshape(-1, 128)
    fm_terms = _gather_all(idx2, u8, i8, f8)
    return _mlp(fm_terms, W1.T, b1.reshape(1, HID), W2.reshape(1, HID),
                b2.reshape(1, 1))


# speedup vs baseline: 1.0833x; 1.0833x over previous
import functools

import jax
import jax.numpy as jnp
from jax import lax
from jax.experimental import pallas as pl
from jax.experimental.pallas import tpu as pltpu
from jax.experimental.pallas import tpu_sc as plsc

B = 16384
F = 26
D = 16
NT = F + 2
CONCAT = NT * D
HID = 256
FIELD_VOCAB = 100000

NC, NS = 2, 16
NW = NC * NS
BPW = B // NW
CH = 128
NCH = BPW // CH


def _gather_body(idx_hbm, user_tbl, item_tbl, feat_tbl, out_hbm,
                 idx_v, rows_v, isem, gsem):
    wid = lax.axis_index("s") * NC + lax.axis_index("c")
    base = wid * BPW
    pltpu.sync_copy(idx_hbm.at[:, pl.ds(base, BPW)], idx_v)
    for f in range(NT):
        tbl = user_tbl if f == 0 else (item_tbl if f == 1 else feat_tbl)
        cps = [
            pltpu.async_copy(
                tbl.at[idx_v.at[f, pl.ds(c * CH, CH)]],
                rows_v.at[pl.ds(c * CH, CH)],
                gsem,
            )
            for c in range(NCH)
        ]
        for cp in cps:
            cp.wait()
        pltpu.sync_copy(rows_v,
                        out_hbm.at[pl.ds(base, BPW), pl.ds(f * D, D)])


@functools.partial(
    pl.kernel,
    out_type=jax.ShapeDtypeStruct((B, CONCAT), jnp.float32),
    mesh=plsc.VectorSubcoreMesh(core_axis_name="c", subcore_axis_name="s"),
    scratch_types=[
        pltpu.VMEM((NT, BPW), jnp.int32),
        pltpu.VMEM((BPW, D), jnp.float32),
        pltpu.SemaphoreType.DMA,
        pltpu.SemaphoreType.DMA,
    ],
    compiler_params=pltpu.CompilerParams(use_tc_tiling_on_sc=False),
)
def _gather_all(idx, user_tbl, item_tbl, feat_tbl, out,
                idx_v, rows_v, isem, gsem):
    _gather_body(idx, user_tbl, item_tbl, feat_tbl, out,
                 idx_v, rows_v, isem, gsem)


BLK = 512


def _mlp_body(x_ref, w1t_ref, b1_ref, w2_ref, b2_ref, o_ref):
    x = x_ref[...]
    h = jnp.dot(x, w1t_ref[...], preferred_element_type=jnp.float32)
    h = jnp.maximum(h + b1_ref[...], 0.0)
    d = jnp.sum(h * w2_ref[...], axis=1, keepdims=True)
    fm = jnp.sum(x, axis=1, keepdims=True)
    z = fm + d + b2_ref[...]
    o_ref[...] = 10.0 / (1.0 + jnp.exp(-z))


def _mlp(fm_terms, w1t, b1, w2, b2):
    return pl.pallas_call(
        _mlp_body,
        grid=(B // BLK,),
        in_specs=[
            pl.BlockSpec((BLK, CONCAT), lambda i: (i, 0)),
            pl.BlockSpec((CONCAT, HID), lambda i: (0, 0)),
            pl.BlockSpec((1, HID), lambda i: (0, 0)),
            pl.BlockSpec((1, HID), lambda i: (0, 0)),
            pl.BlockSpec((1, 1), lambda i: (0, 0)),
        ],
        out_specs=pl.BlockSpec((BLK, 1), lambda i: (i, 0)),
        out_shape=jax.ShapeDtypeStruct((B, 1), jnp.float32),
    )(fm_terms, w1t, b1, w2, b2)


def kernel(user, item, feature, user_table, item_table, feat_tables,
           W1, b1, W2, b2):
    offs = jnp.arange(F, dtype=jnp.int32) * FIELD_VOCAB
    idx2 = jnp.concatenate(
        [user.astype(jnp.int32)[None],
         item.astype(jnp.int32)[None],
         (feature.astype(jnp.int32) + offs[None, :]).T], axis=0)
    feat_flat = feat_tables.reshape(F * FIELD_VOCAB, D)
    fm_terms = _gather_all(idx2, user_table, item_table, feat_flat)
    return _mlp(fm_terms, W1.T, b1.reshape(1, HID), W2.reshape(1, HID),
                b2.reshape(1, 1))
